# baseline (speedup 1.0000x reference)
import functools
import jax
import jax.numpy as jnp
from jax.experimental import pallas as pl
from jax.experimental.pallas import tpu as pltpu

K = 5
FIN = 256
FOUT = 256


def _matmul_body(x0_ref, xs_ref, w_ref, o_ref):
    acc = jnp.zeros(o_ref.shape[1:], dtype=jnp.float32)
    for h in range(2):
        acc += jnp.dot(x0_ref[h], w_ref[0, h], preferred_element_type=jnp.float32)
    for km in range(K - 1):
        for h in range(2):
            acc += jnp.dot(xs_ref[km, h], w_ref[km + 1, h],
                           preferred_element_type=jnp.float32)
    o_ref[0] = acc


def _dense_matmul(x0ch, xs_rest, w4, Bv, Mv):
    MT = 2000
    grid = (Bv, Mv // MT)
    return pl.pallas_call(
        _matmul_body,
        grid=grid,
        in_specs=[
            pl.BlockSpec((2, MT, 128), lambda b, i: (b, i, 0)),
            pl.BlockSpec((K - 1, 2, MT, 128), lambda b, i: (0, b, i, 0)),
            pl.BlockSpec((K, 2, 128, FOUT), lambda b, i: (0, 0, 0, 0)),
        ],
        out_specs=pl.BlockSpec((1, MT, FOUT), lambda b, i: (b, i, 0)),
        out_shape=jax.ShapeDtypeStruct((Bv, Mv, FOUT), jnp.float32),
    )(x0ch, xs_rest, w4)


def kernel(input_tensor, L_values, kernel, L_indices):
    Bv, Mv, Fin = input_tensor.shape
    rows = L_indices[0].astype(jnp.int32)
    cols = L_indices[1].astype(jnp.int32)

    x0ch = input_tensor.reshape(Bv, Mv, 2, 128).transpose(0, 2, 1, 3)
    x0ch = x0ch.reshape(2 * Bv, Mv, 128)

    def spmm(z):
        g = z[:, cols, :] * L_values[None, :, None]
        return jax.vmap(lambda gg: jax.ops.segment_sum(gg, rows, num_segments=Mv))(g)

    xk_prev = x0ch
    xk = spmm(x0ch)
    stack = [xk]
    for _ in range(2, K):
        xk, xk_prev = 2.0 * spmm(xk) - xk_prev, xk
        stack.append(xk)
    xs_rest = jnp.stack(stack, axis=0)

    w4 = kernel.reshape(2, 128, K, FOUT).transpose(2, 0, 1, 3)
    return _dense_matmul(x0ch, xs_rest, w4, Bv, Mv)


# speedup vs baseline: 2.7859x; 2.7859x over previous
import functools
import jax
import jax.numpy as jnp
from jax import lax
from jax.experimental import pallas as pl
from jax.experimental.pallas import tpu as pltpu
from jax.experimental.pallas import tpu_sc as plsc

K = 5
FIN = 256
FOUT = 256
M = 10000
HALF = M // 2
NCH = 16
W = 128
NT = 16
EPT = 10240
NG = EPT // 128
APAD = 5120
ROWS_PT = 320
PR = 40


def _sc_spmm(x0flat, colr, rowr2, valr):
    mesh = plsc.VectorSubcoreMesh(core_axis_name="c", subcore_axis_name="s")

    @functools.partial(
        pl.kernel,
        mesh=mesh,
        out_type=jax.ShapeDtypeStruct(((K - 1) * NCH * M, W), jnp.float32),
        scratch_types=[
            pltpu.VMEM((NG, 128), jnp.int32),
            pltpu.VMEM((NG, 128), jnp.int32),
            pltpu.VMEM((NG, 128), jnp.float32),
            pltpu.VMEM((NG, 128), jnp.int32),
            pltpu.VMEM((128, W), jnp.float32),
            pltpu.VMEM((PR, W), jnp.float32),
            pltpu.VMEM((PR, W), jnp.float32),
            pltpu.VMEM((PR, W), jnp.float32),
            pltpu.VMEM_SHARED((APAD, W), jnp.float32),
            pltpu.SemaphoreType.DMA,
        ],
    )
    def spmm_kernel(x0_hbm, col_hbm, row_hbm, val_hbm, xs_hbm,
                    colbuf, rowbuf, valbuf, cbuf2, zbuf, dbuf, pbuf, zbuf0,
                    acc, sem):
        c = lax.axis_index("c")
        s = lax.axis_index("s")
        slab = s * ROWS_PT
        np_s = jnp.where(s == NT - 1, 5, ROWS_PT // PR)

        pltpu.sync_copy(col_hbm.at[s], colbuf)
        pltpu.sync_copy(row_hbm.at[c, s], rowbuf)
        pltpu.sync_copy(val_hbm.at[s], valbuf)

        def zero_body(r, _):
            for w8 in range(W // 16):
                zbuf0[r, pl.ds(w8 * 16, 16)] = jnp.zeros((16,), jnp.float32)
            return _
        lax.fori_loop(0, PR, zero_body, None)

        for k in range(1, K):
            def chunk_body(ch, _, k=k):
                if k == 1:
                    src = x0_hbm
                    off = ch * M
                else:
                    src = xs_hbm
                    off = ((k - 2) * NCH + ch) * M
                offv = jnp.full((16,), off, jnp.int32)

                def z_body(p, _):
                    pltpu.sync_copy(zbuf0, acc.at[pl.ds(slab + p * PR, PR)])
                    return _
                lax.fori_loop(0, np_s, z_body, None)

                def adj_body(g, _):
                    for w8 in range(8):
                        cbuf2[g, pl.ds(w8 * 16, 16)] = (
                            colbuf[g, pl.ds(w8 * 16, 16)] + offv)
                    return _
                lax.fori_loop(0, NG, adj_body, None)

                plsc.subcore_barrier()

                def g_body(g, _):
                    pltpu.async_copy(src.at[cbuf2.at[g]], zbuf, sem).wait()

                    def e_body(e16, _):
                        v16 = valbuf[g, pl.ds(e16 * 16, 16)]
                        for j in range(16):
                            idx = jnp.full((16,), j, jnp.int32)
                            bj = v16.at[idx].get(mode="promise_in_bounds")
                            r = e16 * 16 + j
                            for w8 in range(W // 16):
                                sl = pl.ds(w8 * 16, 16)
                                zbuf[r, sl] = zbuf[r, sl] * bj
                        return _
                    lax.fori_loop(0, 8, e_body, None)

                    pltpu.sync_copy(zbuf, acc.at[rowbuf.at[g]], add=True)
                    return _
                lax.fori_loop(0, NG, g_body, None)

                plsc.subcore_barrier()

                dst_base = ((k - 1) * NCH + ch) * M + c * HALF + slab
                if k == 1:
                    def c_body(p, _):
                        pltpu.sync_copy(
                            acc.at[pl.ds(slab + p * PR, PR)],
                            xs_hbm.at[pl.ds(dst_base + p * PR, PR)])
                        return _
                    lax.fori_loop(0, np_s, c_body, None)
                else:
                    if k == 2:
                        prev_ref = x0_hbm
                        pbase = ch * M + c * HALF + slab
                    else:
                        prev_ref = xs_hbm
                        pbase = ((k - 3) * NCH + ch) * M + c * HALF + slab

                    def d_body(p, _):
                        pltpu.sync_copy(acc.at[pl.ds(slab + p * PR, PR)], dbuf)
                        pltpu.sync_copy(prev_ref.at[pl.ds(pbase + p * PR, PR)],
                                        pbuf)

                        def row_body(r, _):
                            for w8 in range(W // 16):
                                sl = pl.ds(w8 * 16, 16)
                                dbuf[r, sl] = 2.0 * dbuf[r, sl] - pbuf[r, sl]
                            return _
                        lax.fori_loop(0, PR, row_body, None)
                        pltpu.sync_copy(
                            dbuf, xs_hbm.at[pl.ds(dst_base + p * PR, PR)])
                        return _
                    lax.fori_loop(0, np_s, d_body, None)
                return _
            lax.fori_loop(0, NCH, chunk_body, None)

    return spmm_kernel(x0flat, colr, rowr2, valr)


def _matmul_body(x0_ref, xs_ref, w_ref, o_ref):
    acc = jnp.zeros(o_ref.shape[1:], dtype=jnp.float32)
    for h in range(2):
        acc += jnp.dot(x0_ref[h], w_ref[0, h], preferred_element_type=jnp.float32)
    for km in range(K - 1):
        for h in range(2):
            acc += jnp.dot(xs_ref[km, h], w_ref[km + 1, h],
                           preferred_element_type=jnp.float32)
    o_ref[0] = acc


def _dense_matmul(x0ch, xs_rest, w4, Bv, Mv):
    MT = 2000
    grid = (Bv, Mv // MT)
    return pl.pallas_call(
        _matmul_body,
        grid=grid,
        in_specs=[
            pl.BlockSpec((2, MT, W), lambda b, i: (b, i, 0)),
            pl.BlockSpec((K - 1, 2, MT, W), lambda b, i: (0, b, i, 0)),
            pl.BlockSpec((K, 2, W, FOUT), lambda b, i: (0, 0, 0, 0)),
        ],
        out_specs=pl.BlockSpec((1, MT, FOUT), lambda b, i: (b, i, 0)),
        out_shape=jax.ShapeDtypeStruct((Bv, Mv, FOUT), jnp.float32),
    )(x0ch, xs_rest, w4)


def kernel(input_tensor, L_values, kernel, L_indices):
    Bv, Mv, Fin = input_tensor.shape
    rows = L_indices[0].astype(jnp.int32)
    cols = L_indices[1].astype(jnp.int32)
    E = rows.shape[0]

    x0ch = input_tensor.reshape(Bv, Mv, 2, W).transpose(0, 2, 1, 3)
    x0ch = x0ch.reshape(2 * Bv, Mv, W)
    x0flat = x0ch.reshape(2 * Bv * Mv, W)

    EPAD = NT * NG * 128
    pad = EPAD - E
    trash = HALF + (jnp.arange(EPAD, dtype=jnp.int32) % 96)
    colp = jnp.concatenate([cols, jnp.zeros((pad,), jnp.int32)])
    rowp = jnp.concatenate([rows, jnp.full((pad,), -1, jnp.int32)])
    valp = jnp.concatenate([L_values, jnp.zeros((pad,), jnp.float32)])
    rloc = []
    for c in range(2):
        lo, hi = c * HALF, (c + 1) * HALF
        inhalf = (rowp >= lo) & (rowp < hi)
        rloc.append(jnp.where(inhalf, rowp - lo, trash))
    rowr2 = jnp.stack(rloc, 0).reshape(2, NT, NG, 128)
    colr = colp.reshape(NT, NG, 128)
    valr = valp.reshape(NT, NG, 128)

    xsflat = _sc_spmm(x0flat, colr, rowr2, valr)
    xs_rest = xsflat.reshape(K - 1, NCH, Mv, W)

    w4 = kernel.reshape(2, W, K, FOUT).transpose(2, 0, 1, 3)
    return _dense_matmul(x0ch, xs_rest, w4, Bv, Mv)


# speedup vs baseline: 3.1968x; 1.1475x over previous
import functools
import jax
import jax.numpy as jnp
from jax import lax
from jax.experimental import pallas as pl
from jax.experimental.pallas import tpu as pltpu
from jax.experimental.pallas import tpu_sc as plsc

K = 5
FIN = 256
FOUT = 256
M = 10000
HALF = M // 2
NCH = 16
W = 128
NT = 16
EPT = 10240
NG = EPT // 128
APAD = 5120
ROWS_PT = 320
PR = 40


def _sc_spmm(x0flat, colr, rowr2, valr):
    mesh = plsc.VectorSubcoreMesh(core_axis_name="c", subcore_axis_name="s")

    @functools.partial(
        pl.kernel,
        mesh=mesh,
        out_type=jax.ShapeDtypeStruct(((K - 1) * NCH * M, W), jnp.float32),
        scratch_types=[
            pltpu.VMEM((NG, 128), jnp.int32),
            pltpu.VMEM((NG, 128), jnp.int32),
            pltpu.VMEM((NG, 128), jnp.float32),
            pltpu.VMEM((NG, 128), jnp.int32),
            pltpu.VMEM((128, W), jnp.float32),
            pltpu.VMEM((128, W), jnp.float32),
            pltpu.VMEM((PR, W), jnp.float32),
            pltpu.VMEM((PR, W), jnp.float32),
            pltpu.VMEM((PR, W), jnp.float32),
            pltpu.VMEM_SHARED((APAD, W), jnp.float32),
            pltpu.SemaphoreType.DMA,
            pltpu.SemaphoreType.DMA,
        ],
    )
    def spmm_kernel(x0_hbm, col_hbm, row_hbm, val_hbm, xs_hbm,
                    colbuf, rowbuf, valbuf, cbuf2, zb0, zb1,
                    dbuf, pbuf, zbuf0, acc, s0, s1):
        c = lax.axis_index("c")
        s = lax.axis_index("s")
        slab = s * ROWS_PT
        zbs = (zb0, zb1)
        sems = (s0, s1)
        np_s = jnp.where(s == NT - 1, 5, ROWS_PT // PR)

        pltpu.sync_copy(col_hbm.at[s], colbuf)
        pltpu.sync_copy(row_hbm.at[c, s], rowbuf)
        pltpu.sync_copy(val_hbm.at[s], valbuf)

        def zero_body(r, _):
            for w8 in range(W // 16):
                zbuf0[r, pl.ds(w8 * 16, 16)] = jnp.zeros((16,), jnp.float32)
            return _
        lax.fori_loop(0, PR, zero_body, None)

        def accumulate(src, off):
            offv = jnp.full((16,), off, jnp.int32)

            def z_body(p, _):
                pltpu.sync_copy(zbuf0, acc.at[pl.ds(slab + p * PR, PR)])
                return _
            lax.fori_loop(0, np_s, z_body, None)

            def adj_body(g, _):
                for w8 in range(8):
                    cbuf2[g, pl.ds(w8 * 16, 16)] = (
                        colbuf[g, pl.ds(w8 * 16, 16)] + offv)
                return _
            lax.fori_loop(0, NG, adj_body, None)

            plsc.subcore_barrier()

            for i in range(2):
                pltpu.async_copy(src.at[cbuf2.at[i]], zbs[i], sems[i])

            def quad_body(d, _):
                for i in range(2):
                    g = 2 * d + i
                    zb = zbs[i]
                    pltpu.make_async_copy(src.at[cbuf2.at[g]], zb,
                                          sems[i]).wait()

                    def e_body(e16, _, zb=zb, g=g):
                        v16 = valbuf[g, pl.ds(e16 * 16, 16)]
                        for j in range(16):
                            idx = jnp.full((16,), j, jnp.int32)
                            bj = v16.at[idx].get(mode="promise_in_bounds")
                            r = e16 * 16 + j
                            for w8 in range(W // 16):
                                sl = pl.ds(w8 * 16, 16)
                                zb[r, sl] = zb[r, sl] * bj
                        return _
                    lax.fori_loop(0, 8, e_body, None)

                    pltpu.sync_copy(zb, acc.at[rowbuf.at[g]], add=True)

                    @pl.when(g + 2 < NG)
                    def _prefetch(zb=zb, g=g, i=i):
                        pltpu.async_copy(src.at[cbuf2.at[g + 2]], zb, sems[i])
                return _
            lax.fori_loop(0, NG // 2, quad_body, None)

            plsc.subcore_barrier()

        def chunk1_body(ch, _):
            accumulate(x0_hbm, ch * M)
            dst_base = ch * M + c * HALF + slab

            def c_body(p, _):
                pltpu.sync_copy(acc.at[pl.ds(slab + p * PR, PR)],
                                xs_hbm.at[pl.ds(dst_base + p * PR, PR)])
                return _
            lax.fori_loop(0, np_s, c_body, None)
            return _
        lax.fori_loop(0, NCH, chunk1_body, None)

        def step_body(kk, _):
            def chunk_body(ch, _):
                accumulate(xs_hbm, ((kk - 2) * NCH + ch) * M)
                dst_base = ((kk - 1) * NCH + ch) * M + c * HALF + slab
                pbase0 = ch * M + c * HALF + slab
                pbase1 = ((kk - 3) * NCH + ch) * M + c * HALF + slab

                def d_body(p, _):
                    pltpu.sync_copy(acc.at[pl.ds(slab + p * PR, PR)], dbuf)

                    @pl.when(kk == 2)
                    def _p0():
                        pltpu.sync_copy(x0_hbm.at[pl.ds(pbase0 + p * PR, PR)],
                                        pbuf)

                    @pl.when(kk != 2)
                    def _p1():
                        pltpu.sync_copy(xs_hbm.at[pl.ds(pbase1 + p * PR, PR)],
                                        pbuf)

                    def row_body(r, _):
                        for w8 in range(W // 16):
                            sl = pl.ds(w8 * 16, 16)
                            dbuf[r, sl] = 2.0 * dbuf[r, sl] - pbuf[r, sl]
                        return _
                    lax.fori_loop(0, PR, row_body, None)
                    pltpu.sync_copy(
                        dbuf, xs_hbm.at[pl.ds(dst_base + p * PR, PR)])
                    return _
                lax.fori_loop(0, np_s, d_body, None)
                return _
            lax.fori_loop(0, NCH, chunk_body, None)
            return _
        lax.fori_loop(2, K, step_body, None)

    return spmm_kernel(x0flat, colr, rowr2, valr)


def _matmul_body(x0_ref, xs_ref, w_ref, o_ref):
    acc = jnp.zeros(o_ref.shape[1:], dtype=jnp.float32)
    for h in range(2):
        acc += jnp.dot(x0_ref[h], w_ref[0, h], preferred_element_type=jnp.float32)
    for km in range(K - 1):
        for h in range(2):
            acc += jnp.dot(xs_ref[km, h], w_ref[km + 1, h],
                           preferred_element_type=jnp.float32)
    o_ref[0] = acc


def _dense_matmul(x0ch, xs_rest, w4, Bv, Mv):
    MT = 2000
    grid = (Bv, Mv // MT)
    return pl.pallas_call(
        _matmul_body,
        grid=grid,
        in_specs=[
            pl.BlockSpec((2, MT, W), lambda b, i: (b, i, 0)),
            pl.BlockSpec((K - 1, 2, MT, W), lambda b, i: (0, b, i, 0)),
            pl.BlockSpec((K, 2, W, FOUT), lambda b, i: (0, 0, 0, 0)),
        ],
        out_specs=pl.BlockSpec((1, MT, FOUT), lambda b, i: (b, i, 0)),
        out_shape=jax.ShapeDtypeStruct((Bv, Mv, FOUT), jnp.float32),
    )(x0ch, xs_rest, w4)


def kernel(input_tensor, L_values, kernel, L_indices):
    Bv, Mv, Fin = input_tensor.shape
    rows = L_indices[0].astype(jnp.int32)
    cols = L_indices[1].astype(jnp.int32)
    E = rows.shape[0]

    x0ch = input_tensor.reshape(Bv, Mv, 2, W).transpose(0, 2, 1, 3)
    x0ch = x0ch.reshape(2 * Bv, Mv, W)
    x0flat = x0ch.reshape(2 * Bv * Mv, W)

    EPAD = NT * NG * 128
    pad = EPAD - E
    trash = HALF + (jnp.arange(EPAD, dtype=jnp.int32) % 96)
    colp = jnp.concatenate([cols, jnp.zeros((pad,), jnp.int32)])
    rowp = jnp.concatenate([rows, jnp.full((pad,), -1, jnp.int32)])
    valp = jnp.concatenate([L_values, jnp.zeros((pad,), jnp.float32)])
    rloc = []
    for c in range(2):
        lo, hi = c * HALF, (c + 1) * HALF
        inhalf = (rowp >= lo) & (rowp < hi)
        rloc.append(jnp.where(inhalf, rowp - lo, trash))
    rowr2 = jnp.stack(rloc, 0).reshape(2, NT, NG, 128)
    colr = colp.reshape(NT, NG, 128)
    valr = valp.reshape(NT, NG, 128)

    xsflat = _sc_spmm(x0flat, colr, rowr2, valr)
    xs_rest = xsflat.reshape(K - 1, NCH, Mv, W)

    w4 = kernel.reshape(2, W, K, FOUT).transpose(2, 0, 1, 3)
    return _dense_matmul(x0ch, xs_rest, w4, Bv, Mv)
